# baseline (speedup 1.0000x reference)
import jax
import jax.numpy as jnp
from jax import lax
from jax.experimental import pallas as pl
from jax.experimental.pallas import tpu as pltpu
from jax.experimental.pallas import tpu_sc as plsc

N_FEATURES = 26
BATCH = 16384
EMBED_DIM = 32
OUT_DIM = N_FEATURES * EMBED_DIM

_NC, _NS = 2, 16
_NW = _NC * _NS
_B_W = BATCH // _NW
_CHUNK = 128
_N_CHUNK = _B_W // _CHUNK


def _body(*refs):
    idx_refs = refs[:N_FEATURES]
    tab_refs = refs[N_FEATURES:2 * N_FEATURES]
    out = refs[2 * N_FEATURES]
    idx_v = refs[2 * N_FEATURES + 1]
    bufs = refs[2 * N_FEATURES + 2]
    gsem = refs[2 * N_FEATURES + 3]

    wid = lax.axis_index("s") * _NC + lax.axis_index("c")
    base = wid * _B_W
    row0 = wid * _N_CHUNK

    def start_feature(f, slot):
        pltpu.sync_copy(idx_refs[f].at[pl.ds(row0, _N_CHUNK)], idx_v.at[slot])
        descs = []
        for c in range(_N_CHUNK):
            descs.append(pltpu.async_copy(
                tab_refs[f].at[idx_v.at[slot, c]],
                bufs.at[slot, pl.ds(c * _CHUNK, _CHUNK)],
                gsem))
        return descs

    def finish_feature(f, slot, descs):
        for d in descs:
            d.wait()
        pltpu.sync_copy(
            bufs.at[slot],
            out.at[pl.ds(base, _B_W), pl.ds(f * EMBED_DIM, EMBED_DIM)])

    descs = start_feature(0, 0)
    for f in range(1, N_FEATURES):
        next_descs = start_feature(f, f % 2)
        finish_feature(f - 1, (f - 1) % 2, descs)
        descs = next_descs
    finish_feature(N_FEATURES - 1, (N_FEATURES - 1) % 2, descs)


_mesh = plsc.VectorSubcoreMesh(core_axis_name="c", subcore_axis_name="s")

_sc_call = pl.kernel(
    _body,
    out_type=jax.ShapeDtypeStruct((BATCH, OUT_DIM), jnp.float32),
    mesh=_mesh,
    scratch_types=[
        pltpu.VMEM((2, _N_CHUNK, _CHUNK), jnp.int32),
        pltpu.VMEM((2, _B_W, EMBED_DIM), jnp.float32),
        pltpu.SemaphoreType.DMA,
    ],
    compiler_params=pltpu.CompilerParams(use_tc_tiling_on_sc=False),
)


def kernel(*args):
    idxs = [a.reshape(BATCH // _CHUNK, _CHUNK) for a in args[:N_FEATURES]]
    tables = list(args[N_FEATURES:2 * N_FEATURES])
    return _sc_call(*idxs, *tables)
